# Initial kernel scaffold; baseline (speedup 1.0000x reference)
import jax
import jax.numpy as jnp
from jax.experimental import pallas as pl


def kernel(x, token_table, pos_table):
    raise NotImplementedError("write your pallas kernel here")


# speedup vs baseline: 3.2902x; 3.2902x over previous
import functools

import jax
import jax.numpy as jnp
from jax import lax
from jax.experimental import pallas as pl
from jax.experimental.pallas import tpu as pltpu
from jax.experimental.pallas import tpu_sc as plsc

MAXLEN = 200
VOCAB = 100000
EMBED = 64
BATCH = 4096

NC = 2
NS = 16
L = 16
NW = NC * NS
SEQ_PER_W = BATCH // NW
HALF = MAXLEN // 2

_mesh = plsc.VectorSubcoreMesh(
    core_axis_name="c", subcore_axis_name="s", num_cores=NC, num_subcores=NS
)


@functools.partial(
    pl.kernel,
    out_type=jax.ShapeDtypeStruct((BATCH, MAXLEN, EMBED), jnp.float32),
    mesh=_mesh,
    scratch_types=[
        pltpu.VMEM((2 * SEQ_PER_W, HALF), jnp.int32),
        pltpu.VMEM((MAXLEN, EMBED), jnp.float32),
        pltpu.VMEM((MAXLEN, EMBED), jnp.float32),
        pltpu.SemaphoreType.DMA,
    ],
    compiler_params=pltpu.CompilerParams(use_tc_tiling_on_sc=False),
)
def _tok_pos_embed(x_hbm, tok_hbm, pos_hbm, out_hbm, idx_v, pos_v, rows_v, sem):
    wid = lax.axis_index("s") * NC + lax.axis_index("c")
    base = wid * SEQ_PER_W

    pltpu.sync_copy(x_hbm.at[pl.ds(base * 2, 2 * SEQ_PER_W)], idx_v)
    pltpu.sync_copy(pos_hbm, pos_v)

    @pl.loop(0, SEQ_PER_W)
    def _seq(i):
        cp0 = pltpu.async_copy(
            tok_hbm.at[idx_v.at[2 * i]], rows_v.at[pl.ds(0, HALF)], sem
        )
        cp1 = pltpu.async_copy(
            tok_hbm.at[idx_v.at[2 * i + 1]], rows_v.at[pl.ds(HALF, HALF)], sem
        )
        cp0.wait()
        cp1.wait()

        @pl.loop(0, MAXLEN)
        def _row(r):
            for j in range(EMBED // L):
                sl = pl.ds(j * L, L)
                rows_v[r, sl] = rows_v[r, sl] + pos_v[r, sl]

        pltpu.sync_copy(rows_v, out_hbm.at[base + i])


def kernel(x, token_table, pos_table):
    x2 = x.astype(jnp.int32).reshape(2 * BATCH, HALF)
    return _tok_pos_embed(x2, token_table, pos_table)
